# baseline (speedup 1.0000x reference)
import functools

import jax
import jax.numpy as jnp
from jax import lax
from jax.experimental import pallas as pl
from jax.experimental.pallas import tpu as pltpu
from jax.experimental.pallas import tpu_sc as plsc

N_VOCAB = 49408
N_EMBD = 768
N_TOKENS = 77
BATCH = 1024

NC = 2
NS = 16
NW = NC * NS
SEQ_W = BATCH // NW
PT = 80
CH = 16
QN = PT // CH
NCHUNK = SEQ_W * QN
TAIL = 8
NBUF = 4
LANES = 16
G = N_EMBD // LANES
RU = 8


def _embed_body(idx_hbm, table_hbm, pos_hbm, out_hbm, aux_hbm,
                idx_v, pos_v, buf0, buf1, buf2, buf3,
                gs0, gs1, gs2, gs3, ss0, ss1, ss2, ss3):
    bufs = (buf0, buf1, buf2, buf3)
    gsems = (gs0, gs1, gs2, gs3)
    ssems = (ss0, ss1, ss2, ss3)

    cid = lax.axis_index("c")
    sid = lax.axis_index("s")
    wid = sid * NC + cid
    seq0 = wid * SEQ_W

    pltpu.sync_copy(idx_hbm.at[wid], idx_v)
    pltpu.sync_copy(pos_hbm, pos_v)

    def issue_gather(j, b):
        s = j // QN
        q = j - s * QN
        r0 = pl.multiple_of(q * CH, CH)
        iv = idx_v[s, pl.ds(r0, CH)]
        pltpu.async_copy(table_hbm.at[iv], bufs[b], gsems[b])

    def wait_gather(b):
        iv = idx_v[0, pl.ds(0, CH)]
        pltpu.make_async_copy(
            table_hbm.at[iv], bufs[b], gsems[b]).wait()

    def issue_store(j, b):
        s = j // QN
        q = j - s * QN
        seq = seq0 + s

        @pl.when(q < QN - 1)
        def _():
            r0 = pl.multiple_of(q * CH, CH)
            pltpu.async_copy(bufs[b], out_hbm.at[seq, pl.ds(r0, CH)],
                             ssems[b])

        @pl.when(q == QN - 1)
        def _():
            pltpu.async_copy(bufs[b].at[pl.ds(0, TAIL)],
                             out_hbm.at[seq, pl.ds(64, TAIL)], ssems[b])
            pltpu.async_copy(bufs[b].at[pl.ds(TAIL, TAIL)],
                             aux_hbm.at[seq], ssems[b])

    def wait_store(b):
        pltpu.make_async_copy(
            bufs[b], out_hbm.at[0, pl.ds(0, CH)], ssems[b]).wait()

    def add_pos(j, b):
        s = j // QN
        q = j - s * QN
        base = q * CH

        def row_body(to, _):
            for r in range(RU):
                t = to * RU + r
                for g in range(G):
                    sl = pl.ds(g * LANES, LANES)
                    plsc.addupdate(bufs[b].at[t, sl], pos_v[base + t, sl])
            return 0
        lax.fori_loop(0, CH // RU, row_body, 0, unroll=False)

    issue_gather(0, 0)
    issue_gather(1, 1)

    def outer(jo, _):
        for b in range(NBUF):
            j = jo * NBUF + b
            wait_gather(b)
            add_pos(j, b)
            issue_store(j, b)
            bk = (b + 2) % NBUF
            @pl.when(j >= 2)
            def _():
                wait_store(bk)

            @pl.when(j + 2 < NCHUNK)
            def _():
                issue_gather(j + 2, bk)
        return 0

    lax.fori_loop(0, NCHUNK // NBUF, outer, 0, unroll=False)

    for j in range(NCHUNK - 2, NCHUNK):
        wait_store(j % NBUF)


@functools.partial(
    pl.kernel,
    out_type=(jax.ShapeDtypeStruct((BATCH, N_TOKENS, N_EMBD), jnp.float32),
              jax.ShapeDtypeStruct((BATCH, TAIL, N_EMBD), jnp.float32)),
    mesh=plsc.VectorSubcoreMesh(core_axis_name="c", subcore_axis_name="s"),
    scratch_types=[
        pltpu.VMEM((SEQ_W, PT), jnp.int32),
        pltpu.VMEM((PT, N_EMBD), jnp.float32),
        pltpu.VMEM((CH, N_EMBD), jnp.float32),
        pltpu.VMEM((CH, N_EMBD), jnp.float32),
        pltpu.VMEM((CH, N_EMBD), jnp.float32),
        pltpu.VMEM((CH, N_EMBD), jnp.float32),
        pltpu.SemaphoreType.DMA,
        pltpu.SemaphoreType.DMA,
        pltpu.SemaphoreType.DMA,
        pltpu.SemaphoreType.DMA,
        pltpu.SemaphoreType.DMA,
        pltpu.SemaphoreType.DMA,
        pltpu.SemaphoreType.DMA,
        pltpu.SemaphoreType.DMA,
    ],
)
def _embed_kernel(idx_hbm, table_hbm, pos_hbm, out_hbm, aux_hbm, *scratch):
    _embed_body(idx_hbm, table_hbm, pos_hbm, out_hbm, aux_hbm, *scratch)


def kernel(tokens, token_embedding, position_embedding):
    tok = jnp.pad(jnp.asarray(tokens, jnp.int32),
                  ((0, 0), (0, PT - N_TOKENS)))
    idx = tok.reshape(NW, SEQ_W, PT)
    pos = jnp.pad(position_embedding, ((0, PT - N_TOKENS), (0, 0)))
    main, aux = _embed_kernel(idx, token_embedding, pos)
    return lax.dynamic_update_slice(main, aux[:, :N_TOKENS - 72, :],
                                    (0, 72, 0))
